# Initial kernel scaffold; baseline (speedup 1.0000x reference)
import jax
import jax.numpy as jnp
from jax.experimental import pallas as pl


def kernel(h_s, Q_t, etype, tau_s, tau_t, dt, rte_emb, rte_lin_W, rte_lin_b, K_W, K_b, V_W, V_b, Watt_W, Wmsg_W, mu):
    raise NotImplementedError("write your pallas kernel here")


# speedup vs baseline: 1.1059x; 1.1059x over previous
import functools
import math

import jax
import jax.numpy as jnp
from jax.experimental import pallas as pl
from jax.experimental.pallas import tpu as pltpu

E_BLOCK = 640


def _table_kernel(emb_ref, w_ref, b_ref, out_ref):
    out_ref[...] = (
        jnp.dot(emb_ref[...], w_ref[...].T, preferred_element_type=jnp.float32)
        + b_ref[...]
    )


def _hgt_kernel(idx_ref, hs_ref, q_ref, table_ref, kcat_ref, vcat_ref,
                kb_ref, vb_ref, wmsg_ref, watt_ref, mu_ref,
                att_ref, m_ref, *, maxlen, nn, ne, h, dk):
    B = hs_ref.shape[0]
    dt = idx_ref[:, 0:1]
    tau = idx_ref[:, 1:2]
    et = idx_ref[:, 2:3]

    iota_dt = jax.lax.broadcasted_iota(jnp.int32, (B, maxlen), 1)
    oh = (dt == iota_dt).astype(jnp.bfloat16)
    hh = hs_ref[...] + jnp.dot(oh, table_ref[...],
                               preferred_element_type=jnp.float32)
    hh16 = hh.astype(jnp.bfloat16)

    kfull = jnp.dot(hh16, kcat_ref[...], preferred_element_type=jnp.float32)
    vfull = jnp.dot(hh16, vcat_ref[...], preferred_element_type=jnp.float32)
    d = hs_ref.shape[1]
    ks = jnp.zeros((B, d), jnp.float32)
    vs = jnp.zeros((B, d), jnp.float32)
    for t in range(nn):
        mt = tau == t
        ks = jnp.where(mt, kfull[:, t * d:(t + 1) * d] + kb_ref[t], ks)
        vs = jnp.where(mt, vfull[:, t * d:(t + 1) * d] + vb_ref[t], vs)
    ks16 = ks.astype(jnp.bfloat16)
    vs16 = vs.astype(jnp.bfloat16)

    att_cols = []
    for hd in range(h):
        sl = slice(dk * hd, dk * (hd + 1))
        ym = jnp.dot(vs16[:, sl], wmsg_ref[...],
                     preferred_element_type=jnp.float32)
        ya = jnp.dot(ks16[:, sl], watt_ref[...],
                     preferred_element_type=jnp.float32)
        msel = jnp.zeros((B, dk), jnp.float32)
        asel = jnp.zeros((B, dk), jnp.float32)
        for t in range(ne):
            mt = et == t
            msel = jnp.where(mt, ym[:, dk * t:dk * (t + 1)], msel)
            asel = jnp.where(mt, ya[:, dk * t:dk * (t + 1)], asel)
        m_ref[:, sl] = msel
        att_cols.append(
            jnp.sum(asel * q_ref[:, sl], axis=1, keepdims=True))

    att = jnp.concatenate(att_cols, axis=1)
    musel = jnp.zeros((B, h), jnp.float32)
    for t in range(ne):
        musel = jnp.where(et == t, mu_ref[t], musel)
    att_ref[...] = att * musel * (1.0 / math.sqrt(dk))


def kernel(h_s, Q_t, etype, tau_s, tau_t, dt, rte_emb, rte_lin_W, rte_lin_b,
           K_W, K_b, V_W, V_b, Watt_W, Wmsg_W, mu):
    e, d_in = h_s.shape
    h, dk = Q_t.shape[1], Q_t.shape[2]
    maxlen = rte_emb.shape[0]
    nn = K_W.shape[0]
    ne = Watt_W.shape[0]
    d_out = K_W.shape[1]

    table = pl.pallas_call(
        _table_kernel,
        out_shape=jax.ShapeDtypeStruct((maxlen, d_in), jnp.float32),
    )(rte_emb, rte_lin_W, rte_lin_b.reshape(1, d_in))
    table16 = table.astype(jnp.bfloat16)

    kcat = jnp.transpose(K_W, (2, 0, 1)).reshape(d_in, nn * d_out)
    vcat = jnp.transpose(V_W, (2, 0, 1)).reshape(d_in, nn * d_out)
    wmsg = jnp.transpose(Wmsg_W, (2, 0, 1)).reshape(dk, ne * dk)
    watt = jnp.transpose(Watt_W, (2, 0, 1)).reshape(dk, ne * dk)
    kcat16, vcat16 = kcat.astype(jnp.bfloat16), vcat.astype(jnp.bfloat16)
    wmsg16, watt16 = wmsg.astype(jnp.bfloat16), watt.astype(jnp.bfloat16)

    idx = jnp.stack([dt, tau_s, etype], axis=1).astype(jnp.int32)
    q2 = Q_t.reshape(e, h * dk)

    ep = ((e + E_BLOCK - 1) // E_BLOCK) * E_BLOCK
    if ep != e:
        pad = ep - e
        idx = jnp.pad(idx, ((0, pad), (0, 0)))
        h_sp = jnp.pad(h_s, ((0, pad), (0, 0)))
        q2 = jnp.pad(q2, ((0, pad), (0, 0)))
    else:
        h_sp = h_s
    nb = ep // E_BLOCK

    body = functools.partial(_hgt_kernel, maxlen=maxlen, nn=nn, ne=ne,
                             h=h, dk=dk)
    row_spec = lambda w: pl.BlockSpec((E_BLOCK, w), lambda i: (i, 0))
    full_spec = lambda a: pl.BlockSpec(a.shape, lambda i: (0,) * a.ndim)
    att, m = pl.pallas_call(
        body,
        grid=(nb,),
        in_specs=[
            row_spec(3),
            row_spec(d_in),
            row_spec(h * dk),
            full_spec(table16),
            full_spec(kcat16),
            full_spec(vcat16),
            full_spec(K_b),
            full_spec(V_b),
            full_spec(wmsg16),
            full_spec(watt16),
            full_spec(mu),
        ],
        out_specs=[
            pl.BlockSpec((E_BLOCK, h), lambda i: (i, 0)),
            pl.BlockSpec((E_BLOCK, h * dk), lambda i: (i, 0)),
        ],
        out_shape=[
            jax.ShapeDtypeStruct((ep, h), jnp.float32),
            jax.ShapeDtypeStruct((ep, h * dk), jnp.float32),
        ],
        compiler_params=pltpu.CompilerParams(
            dimension_semantics=("parallel",)),
    )(idx, h_sp, q2, table16, kcat16, vcat16, K_b, V_b, wmsg16, watt16, mu)

    return att[:e], m[:e].reshape(e, h, dk)


# speedup vs baseline: 4.2345x; 3.8288x over previous
import functools
import math

import jax
import jax.numpy as jnp
from jax.experimental import pallas as pl
from jax.experimental.pallas import tpu as pltpu

E_BLOCK = 640


def _table_kernel(emb_ref, w_ref, b_ref, out_ref):
    out_ref[...] = (
        jnp.dot(emb_ref[...], w_ref[...].T, preferred_element_type=jnp.float32)
        + b_ref[...]
    )


def _hgt_kernel(idx_ref, hs_ref, q_ref, table_ref, kcat_ref, vcat_ref,
                kb_ref, vb_ref, wmsgbd_ref, wattbd_ref, mu_ref, seg_ref,
                att_ref, m_ref, *, maxlen, nn, ne, h, dk):
    B = hs_ref.shape[0]
    dt = idx_ref[:, 0:1]
    tau = idx_ref[:, 1:2]
    et = idx_ref[:, 2:3]

    iota_dt = jax.lax.broadcasted_iota(jnp.int32, (B, maxlen), 1)
    oh = (dt == iota_dt).astype(jnp.bfloat16)
    hh = hs_ref[...] + jnp.dot(oh, table_ref[...],
                               preferred_element_type=jnp.float32)
    hh16 = hh.astype(jnp.bfloat16)

    kfull = jnp.dot(hh16, kcat_ref[...], preferred_element_type=jnp.float32)
    vfull = jnp.dot(hh16, vcat_ref[...], preferred_element_type=jnp.float32)
    d = hs_ref.shape[1]
    ks = jnp.zeros((B, d), jnp.float32)
    vs = jnp.zeros((B, d), jnp.float32)
    for t in range(nn):
        mt = tau == t
        ks = jnp.where(mt, kfull[:, t * d:(t + 1) * d] + kb_ref[t], ks)
        vs = jnp.where(mt, vfull[:, t * d:(t + 1) * d] + vb_ref[t], vs)
    ks16 = ks.astype(jnp.bfloat16)
    vs16 = vs.astype(jnp.bfloat16)

    ymf = jnp.dot(vs16, wmsgbd_ref[...], preferred_element_type=jnp.float32)
    yaf = jnp.dot(ks16, wattbd_ref[...], preferred_element_type=jnp.float32)
    msel = jnp.zeros((B, d), jnp.float32)
    asel = jnp.zeros((B, d), jnp.float32)
    musel = jnp.zeros((B, h), jnp.float32)
    for t in range(ne):
        mt = et == t
        msel = jnp.where(mt, ymf[:, t * d:(t + 1) * d], msel)
        asel = jnp.where(mt, yaf[:, t * d:(t + 1) * d], asel)
        musel = jnp.where(mt, mu_ref[t], musel)
    m_ref[...] = msel

    prod = (asel * q_ref[...]).astype(jnp.bfloat16)
    att = jnp.dot(prod, seg_ref[...], preferred_element_type=jnp.float32)
    att_ref[...] = att * musel * (1.0 / math.sqrt(dk))


def kernel(h_s, Q_t, etype, tau_s, tau_t, dt, rte_emb, rte_lin_W, rte_lin_b,
           K_W, K_b, V_W, V_b, Watt_W, Wmsg_W, mu):
    e, d_in = h_s.shape
    h, dk = Q_t.shape[1], Q_t.shape[2]
    maxlen = rte_emb.shape[0]
    nn = K_W.shape[0]
    ne = Watt_W.shape[0]
    d_out = K_W.shape[1]

    table = pl.pallas_call(
        _table_kernel,
        out_shape=jax.ShapeDtypeStruct((maxlen, d_in), jnp.float32),
    )(rte_emb, rte_lin_W, rte_lin_b.reshape(1, d_in))
    table16 = table.astype(jnp.bfloat16)

    kcat = jnp.transpose(K_W, (2, 0, 1)).reshape(d_in, nn * d_out)
    vcat = jnp.transpose(V_W, (2, 0, 1)).reshape(d_in, nn * d_out)
    eye_h = jnp.eye(h, dtype=jnp.float32)
    wmsgbd = jnp.concatenate(
        [jnp.kron(eye_h, Wmsg_W[t].T) for t in range(ne)], axis=1)
    wattbd = jnp.concatenate(
        [jnp.kron(eye_h, Watt_W[t].T) for t in range(ne)], axis=1)
    seg = jnp.kron(jnp.eye(h, dtype=jnp.bfloat16),
                   jnp.ones((dk, 1), dtype=jnp.bfloat16))
    kcat16, vcat16 = kcat.astype(jnp.bfloat16), vcat.astype(jnp.bfloat16)
    wmsg16, watt16 = wmsgbd.astype(jnp.bfloat16), wattbd.astype(jnp.bfloat16)

    idx = jnp.stack([dt, tau_s, etype], axis=1).astype(jnp.int32)
    q2 = Q_t.reshape(e, h * dk)

    ep = ((e + E_BLOCK - 1) // E_BLOCK) * E_BLOCK
    if ep != e:
        pad = ep - e
        idx = jnp.pad(idx, ((0, pad), (0, 0)))
        h_sp = jnp.pad(h_s, ((0, pad), (0, 0)))
        q2 = jnp.pad(q2, ((0, pad), (0, 0)))
    else:
        h_sp = h_s
    nb = ep // E_BLOCK

    body = functools.partial(_hgt_kernel, maxlen=maxlen, nn=nn, ne=ne,
                             h=h, dk=dk)
    row_spec = lambda w: pl.BlockSpec((E_BLOCK, w), lambda i: (i, 0))
    full_spec = lambda a: pl.BlockSpec(a.shape, lambda i: (0,) * a.ndim)
    att, m = pl.pallas_call(
        body,
        grid=(nb,),
        in_specs=[
            row_spec(3),
            row_spec(d_in),
            row_spec(h * dk),
            full_spec(table16),
            full_spec(kcat16),
            full_spec(vcat16),
            full_spec(K_b),
            full_spec(V_b),
            full_spec(wmsg16),
            full_spec(watt16),
            full_spec(mu),
            full_spec(seg),
        ],
        out_specs=[
            pl.BlockSpec((E_BLOCK, h), lambda i: (i, 0)),
            pl.BlockSpec((E_BLOCK, h * dk), lambda i: (i, 0)),
        ],
        out_shape=[
            jax.ShapeDtypeStruct((ep, h), jnp.float32),
            jax.ShapeDtypeStruct((ep, h * dk), jnp.float32),
        ],
        compiler_params=pltpu.CompilerParams(
            dimension_semantics=("parallel",)),
    )(idx, h_sp, q2, table16, kcat16, vcat16, K_b, V_b, wmsg16, watt16, mu,
      seg)

    return att[:e], m[:e].reshape(e, h, dk)


# speedup vs baseline: 4.6706x; 1.1030x over previous
import functools
import math

import jax
import jax.numpy as jnp
from jax.experimental import pallas as pl
from jax.experimental.pallas import tpu as pltpu

E_BLOCK = 640


def _table_kernel(emb_ref, w_ref, b_ref, out_ref):
    out_ref[...] = (
        jnp.dot(emb_ref[...], w_ref[...].T, preferred_element_type=jnp.float32)
        + b_ref[...]
    )


def _dgt(w, x, prec=jnp.float32):
    return jax.lax.dot_general(w, x, (((0,), (0,)), ((), ())),
                               preferred_element_type=prec)


def _hgt_kernel(idx_ref, hst_ref, qt_ref, table_ref, kcat_ref, vcat_ref,
                kb_ref, vb_ref, wmsgbd_ref, wattbd_ref, mu_ref, seg_ref,
                att_ref, m_ref, *, maxlen, nn, ne, h, dk):
    B = hst_ref.shape[1]
    d = hst_ref.shape[0]
    dt = idx_ref[0:1, :]
    tau = idx_ref[1:2, :]
    et = idx_ref[2:3, :]

    iota_dt = jax.lax.broadcasted_iota(jnp.int32, (maxlen, B), 0)
    oht = (dt == iota_dt).astype(jnp.bfloat16)
    hht = hst_ref[...] + _dgt(table_ref[...], oht)
    hht16 = hht.astype(jnp.bfloat16)

    kfull = _dgt(kcat_ref[...], hht16)
    vfull = _dgt(vcat_ref[...], hht16)
    oh_nn = (tau == jax.lax.broadcasted_iota(jnp.int32, (nn, B), 0)
             ).astype(jnp.float32)
    kbias = _dgt(kb_ref[...], oh_nn)
    vbias = _dgt(vb_ref[...], oh_nn)
    ks = jnp.zeros((d, B), jnp.float32)
    vs = jnp.zeros((d, B), jnp.float32)
    for t in range(nn):
        mt = tau == t
        ks = jnp.where(mt, kfull[t * d:(t + 1) * d, :], ks)
        vs = jnp.where(mt, vfull[t * d:(t + 1) * d, :], vs)
    ks16 = (ks + kbias).astype(jnp.bfloat16)
    vs16 = (vs + vbias).astype(jnp.bfloat16)

    ymf = _dgt(wmsgbd_ref[...], vs16)
    yaf = _dgt(wattbd_ref[...], ks16)
    msel = jnp.zeros((d, B), jnp.float32)
    asel = jnp.zeros((d, B), jnp.float32)
    for t in range(ne):
        mt = et == t
        msel = jnp.where(mt, ymf[t * d:(t + 1) * d, :], msel)
        asel = jnp.where(mt, yaf[t * d:(t + 1) * d, :], asel)
    m_ref[...] = msel

    prod = (asel * qt_ref[...]).astype(jnp.bfloat16)
    att = _dgt(seg_ref[...], prod)
    oh_ne = (et == jax.lax.broadcasted_iota(jnp.int32, (ne, B), 0)
             ).astype(jnp.float32)
    musel = _dgt(mu_ref[...], oh_ne)
    att_ref[...] = att * musel * (1.0 / math.sqrt(dk))


def kernel(h_s, Q_t, etype, tau_s, tau_t, dt, rte_emb, rte_lin_W, rte_lin_b,
           K_W, K_b, V_W, V_b, Watt_W, Wmsg_W, mu):
    e, d_in = h_s.shape
    h, dk = Q_t.shape[1], Q_t.shape[2]
    maxlen = rte_emb.shape[0]
    nn = K_W.shape[0]
    ne = Watt_W.shape[0]
    d_out = K_W.shape[1]

    table = pl.pallas_call(
        _table_kernel,
        out_shape=jax.ShapeDtypeStruct((maxlen, d_in), jnp.float32),
    )(rte_emb, rte_lin_W, rte_lin_b.reshape(1, d_in))
    table16 = table.astype(jnp.bfloat16)

    kcat = jnp.transpose(K_W, (2, 0, 1)).reshape(d_in, nn * d_out)
    vcat = jnp.transpose(V_W, (2, 0, 1)).reshape(d_in, nn * d_out)
    eye_h = jnp.eye(h, dtype=jnp.float32)
    wmsgbd = jnp.concatenate(
        [jnp.kron(eye_h, Wmsg_W[t].T) for t in range(ne)], axis=1)
    wattbd = jnp.concatenate(
        [jnp.kron(eye_h, Watt_W[t].T) for t in range(ne)], axis=1)
    seg = jnp.kron(jnp.eye(h, dtype=jnp.bfloat16),
                   jnp.ones((dk, 1), dtype=jnp.bfloat16))
    kcat16, vcat16 = kcat.astype(jnp.bfloat16), vcat.astype(jnp.bfloat16)
    wmsg16, watt16 = wmsgbd.astype(jnp.bfloat16), wattbd.astype(jnp.bfloat16)
    hst = h_s.T
    qt = Q_t.transpose(1, 2, 0).reshape(h * dk, e)
    idx3 = jnp.stack([dt, tau_s, etype], axis=0).astype(jnp.int32)

    ep = ((e + E_BLOCK - 1) // E_BLOCK) * E_BLOCK
    if ep != e:
        pad = ep - e
        idx3 = jnp.pad(idx3, ((0, 0), (0, pad)))
        hst = jnp.pad(hst, ((0, 0), (0, pad)))
        qt = jnp.pad(qt, ((0, 0), (0, pad)))
    nb = ep // E_BLOCK

    body = functools.partial(_hgt_kernel, maxlen=maxlen, nn=nn, ne=ne,
                             h=h, dk=dk)
    col_spec = lambda r: pl.BlockSpec((r, E_BLOCK), lambda i: (0, i))
    full_spec = lambda a: pl.BlockSpec(a.shape, lambda i: (0,) * a.ndim)
    att_t, m_t = pl.pallas_call(
        body,
        grid=(nb,),
        in_specs=[
            col_spec(3),
            col_spec(d_in),
            col_spec(h * dk),
            full_spec(table16),
            full_spec(kcat16),
            full_spec(vcat16),
            full_spec(K_b),
            full_spec(V_b),
            full_spec(wmsg16),
            full_spec(watt16),
            full_spec(mu),
            full_spec(seg),
        ],
        out_specs=[
            pl.BlockSpec((h, E_BLOCK), lambda i: (0, i)),
            pl.BlockSpec((d_out, E_BLOCK), lambda i: (0, i)),
        ],
        out_shape=[
            jax.ShapeDtypeStruct((h, ep), jnp.float32),
            jax.ShapeDtypeStruct((d_out, ep), jnp.float32),
        ],
        compiler_params=pltpu.CompilerParams(
            dimension_semantics=("parallel",)),
    )(idx3, hst, qt, table16, kcat16, vcat16, K_b, V_b, wmsg16, watt16, mu,
      seg)

    att = att_t[:, :e].T
    m = m_t[:, :e].reshape(h, dk, e).transpose(2, 0, 1)
    return att, m


# speedup vs baseline: 4.7101x; 1.0084x over previous
import functools
import math

import jax
import jax.numpy as jnp
from jax.experimental import pallas as pl
from jax.experimental.pallas import tpu as pltpu

E_BLOCK = 640


def _table_kernel(emb_ref, w_ref, b_ref, out_ref):
    out_ref[...] = (
        jnp.dot(w_ref[...], emb_ref[...].T, preferred_element_type=jnp.float32)
        + b_ref[...]
    )


def _mm(w, x):
    return jnp.dot(w, x, preferred_element_type=jnp.float32)


def _hgt_kernel(idx_ref, hst_ref, qt_ref, table_ref, kcat_ref, vcat_ref,
                kb_ref, vb_ref, wmsgbd_ref, wattbd_ref, mu_ref, seg_ref,
                att_ref, m_ref, *, maxlen, nn, ne, h, dk):
    B = hst_ref.shape[1]
    d = hst_ref.shape[0]
    dt = idx_ref[0:1, :]
    tau = idx_ref[1:2, :]
    et = idx_ref[2:3, :]

    iota_dt = jax.lax.broadcasted_iota(jnp.int32, (maxlen, B), 0)
    oht = (dt == iota_dt).astype(jnp.bfloat16)
    hht = hst_ref[...] + _mm(table_ref[...], oht)
    hht16 = hht.astype(jnp.bfloat16)

    kfull = _mm(kcat_ref[...], hht16)
    vfull = _mm(vcat_ref[...], hht16)
    oh_nn = (tau == jax.lax.broadcasted_iota(jnp.int32, (nn, B), 0)
             ).astype(jnp.float32)
    kbias = _mm(kb_ref[...], oh_nn)
    vbias = _mm(vb_ref[...], oh_nn)
    ks = jnp.zeros((d, B), jnp.float32)
    vs = jnp.zeros((d, B), jnp.float32)
    for t in range(nn):
        mt = tau == t
        ks = jnp.where(mt, kfull[t * d:(t + 1) * d, :], ks)
        vs = jnp.where(mt, vfull[t * d:(t + 1) * d, :], vs)
    ks16 = (ks + kbias).astype(jnp.bfloat16)
    vs16 = (vs + vbias).astype(jnp.bfloat16)

    ymf = _mm(wmsgbd_ref[...], vs16)
    yaf = _mm(wattbd_ref[...], ks16)
    msel = jnp.zeros((d, B), jnp.float32)
    asel = jnp.zeros((d, B), jnp.float32)
    for t in range(ne):
        mt = et == t
        msel = jnp.where(mt, ymf[t * d:(t + 1) * d, :], msel)
        asel = jnp.where(mt, yaf[t * d:(t + 1) * d, :], asel)
    m_ref[...] = msel

    prod = (asel * qt_ref[...]).astype(jnp.bfloat16)
    att = _mm(seg_ref[...], prod)
    oh_ne = (et == jax.lax.broadcasted_iota(jnp.int32, (ne, B), 0)
             ).astype(jnp.float32)
    musel = _mm(mu_ref[...], oh_ne)
    att_ref[...] = att * musel * (1.0 / math.sqrt(dk))


def kernel(h_s, Q_t, etype, tau_s, tau_t, dt, rte_emb, rte_lin_W, rte_lin_b,
           K_W, K_b, V_W, V_b, Watt_W, Wmsg_W, mu):
    e, d_in = h_s.shape
    h, dk = Q_t.shape[1], Q_t.shape[2]
    maxlen = rte_emb.shape[0]
    nn = K_W.shape[0]
    ne = Watt_W.shape[0]
    d_out = K_W.shape[1]

    table_t = pl.pallas_call(
        _table_kernel,
        out_shape=jax.ShapeDtypeStruct((d_in, maxlen), jnp.float32),
    )(rte_emb, rte_lin_W, rte_lin_b.reshape(d_in, 1))
    table16 = table_t.astype(jnp.bfloat16)

    kcat = K_W.reshape(nn * d_out, d_in)
    vcat = V_W.reshape(nn * d_out, d_in)
    eye_h = jnp.eye(h, dtype=jnp.float32)
    wmsgbd = jnp.concatenate(
        [jnp.kron(eye_h, Wmsg_W[t]) for t in range(ne)], axis=0)
    wattbd = jnp.concatenate(
        [jnp.kron(eye_h, Watt_W[t]) for t in range(ne)], axis=0)
    seg = jnp.kron(jnp.eye(h, dtype=jnp.bfloat16),
                   jnp.ones((1, dk), dtype=jnp.bfloat16))
    kcat16, vcat16 = kcat.astype(jnp.bfloat16), vcat.astype(jnp.bfloat16)
    wmsg16, watt16 = wmsgbd.astype(jnp.bfloat16), wattbd.astype(jnp.bfloat16)
    kb_t, vb_t, mu_t = K_b.T, V_b.T, mu.T
    hst = h_s.T
    qt = Q_t.transpose(1, 2, 0).reshape(h * dk, e)
    idx3 = jnp.stack([dt, tau_s, etype], axis=0).astype(jnp.int32)

    ep = ((e + E_BLOCK - 1) // E_BLOCK) * E_BLOCK
    if ep != e:
        pad = ep - e
        idx3 = jnp.pad(idx3, ((0, 0), (0, pad)))
        hst = jnp.pad(hst, ((0, 0), (0, pad)))
        qt = jnp.pad(qt, ((0, 0), (0, pad)))
    nb = ep // E_BLOCK

    body = functools.partial(_hgt_kernel, maxlen=maxlen, nn=nn, ne=ne,
                             h=h, dk=dk)
    col_spec = lambda r: pl.BlockSpec((r, E_BLOCK), lambda i: (0, i))
    full_spec = lambda a: pl.BlockSpec(a.shape, lambda i: (0,) * a.ndim)
    att_t, m_t = pl.pallas_call(
        body,
        grid=(nb,),
        in_specs=[
            col_spec(3),
            col_spec(d_in),
            col_spec(h * dk),
            full_spec(table16),
            full_spec(kcat16),
            full_spec(vcat16),
            full_spec(kb_t),
            full_spec(vb_t),
            full_spec(wmsg16),
            full_spec(watt16),
            full_spec(mu_t),
            full_spec(seg),
        ],
        out_specs=[
            pl.BlockSpec((h, E_BLOCK), lambda i: (0, i)),
            pl.BlockSpec((d_out, E_BLOCK), lambda i: (0, i)),
        ],
        out_shape=[
            jax.ShapeDtypeStruct((h, ep), jnp.float32),
            jax.ShapeDtypeStruct((d_out, ep), jnp.float32),
        ],
        compiler_params=pltpu.CompilerParams(
            dimension_semantics=("parallel",)),
    )(idx3, hst, qt, table16, kcat16, vcat16, kb_t, vb_t, wmsg16, watt16,
      mu_t, seg)

    att = att_t[:, :e].T
    m = m_t[:, :e].reshape(h, dk, e).transpose(2, 0, 1)
    return att, m


# speedup vs baseline: 5.2006x; 1.1041x over previous
import functools
import math

import jax
import jax.numpy as jnp
from jax.experimental import pallas as pl
from jax.experimental.pallas import tpu as pltpu

E_BLOCK = 640


def _table_kernel(emb_ref, w_ref, b_ref, out_ref):
    out_ref[...] = (
        jnp.dot(w_ref[...], emb_ref[...].T, preferred_element_type=jnp.float32)
        + b_ref[...]
    )


def _mm(w, x):
    return jnp.dot(w, x, preferred_element_type=jnp.float32)


def _hgt_kernel(idx_ref, hs_ref, qt_ref, table_ref, kcat_ref, vcat_ref,
                kb_ref, vb_ref, wmsgbd_ref, wattbd_ref, mu_ref, seg_ref,
                att_ref, m_ref, *, maxlen, nn, ne, h, dk):
    B = hs_ref.shape[0]
    d = hs_ref.shape[1]
    dt = idx_ref[0:1, :]
    tau = idx_ref[1:2, :]
    et = idx_ref[2:3, :]

    iota_dt = jax.lax.broadcasted_iota(jnp.int32, (maxlen, B), 0)
    oht = (dt == iota_dt).astype(jnp.bfloat16)
    hst = hs_ref[...].astype(jnp.bfloat16).T.astype(jnp.float32)
    hht = hst + _mm(table_ref[...], oht)
    hht16 = hht.astype(jnp.bfloat16)

    kfull = _mm(kcat_ref[...], hht16)
    vfull = _mm(vcat_ref[...], hht16)
    oh_nn = (tau == jax.lax.broadcasted_iota(jnp.int32, (nn, B), 0)
             ).astype(jnp.float32)
    kbias = _mm(kb_ref[...], oh_nn)
    vbias = _mm(vb_ref[...], oh_nn)
    ks = jnp.zeros((d, B), jnp.float32)
    vs = jnp.zeros((d, B), jnp.float32)
    for t in range(nn):
        mt = tau == t
        ks = jnp.where(mt, kfull[t * d:(t + 1) * d, :], ks)
        vs = jnp.where(mt, vfull[t * d:(t + 1) * d, :], vs)
    ks16 = (ks + kbias).astype(jnp.bfloat16)
    vs16 = (vs + vbias).astype(jnp.bfloat16)

    ymf = _mm(wmsgbd_ref[...], vs16)
    yaf = _mm(wattbd_ref[...], ks16)
    msel = jnp.zeros((d, B), jnp.float32)
    asel = jnp.zeros((d, B), jnp.float32)
    for t in range(ne):
        mt = et == t
        msel = jnp.where(mt, ymf[t * d:(t + 1) * d, :], msel)
        asel = jnp.where(mt, yaf[t * d:(t + 1) * d, :], asel)
    m_ref[...] = msel

    prod = (asel * qt_ref[...]).astype(jnp.bfloat16)
    att = _mm(seg_ref[...], prod)
    oh_ne = (et == jax.lax.broadcasted_iota(jnp.int32, (ne, B), 0)
             ).astype(jnp.float32)
    musel = _mm(mu_ref[...], oh_ne)
    att_ref[...] = att * musel * (1.0 / math.sqrt(dk))


def kernel(h_s, Q_t, etype, tau_s, tau_t, dt, rte_emb, rte_lin_W, rte_lin_b,
           K_W, K_b, V_W, V_b, Watt_W, Wmsg_W, mu):
    e, d_in = h_s.shape
    h, dk = Q_t.shape[1], Q_t.shape[2]
    maxlen = rte_emb.shape[0]
    nn = K_W.shape[0]
    ne = Watt_W.shape[0]
    d_out = K_W.shape[1]

    table_t = pl.pallas_call(
        _table_kernel,
        out_shape=jax.ShapeDtypeStruct((d_in, maxlen), jnp.float32),
    )(rte_emb, rte_lin_W, rte_lin_b.reshape(d_in, 1))
    table16 = table_t.astype(jnp.bfloat16)

    kcat = K_W.reshape(nn * d_out, d_in)
    vcat = V_W.reshape(nn * d_out, d_in)
    eye_h = jnp.eye(h, dtype=jnp.float32)
    wmsgbd = jnp.concatenate(
        [jnp.kron(eye_h, Wmsg_W[t]) for t in range(ne)], axis=0)
    wattbd = jnp.concatenate(
        [jnp.kron(eye_h, Watt_W[t]) for t in range(ne)], axis=0)
    seg = jnp.kron(jnp.eye(h, dtype=jnp.bfloat16),
                   jnp.ones((1, dk), dtype=jnp.bfloat16))
    kcat16, vcat16 = kcat.astype(jnp.bfloat16), vcat.astype(jnp.bfloat16)
    wmsg16, watt16 = wmsgbd.astype(jnp.bfloat16), wattbd.astype(jnp.bfloat16)
    kb_t, vb_t, mu_t = K_b.T, V_b.T, mu.T
    qt = Q_t.transpose(1, 2, 0).reshape(h * dk, e)
    idx3 = jnp.stack([dt, tau_s, etype], axis=0).astype(jnp.int32)

    ep = ((e + E_BLOCK - 1) // E_BLOCK) * E_BLOCK
    if ep != e:
        pad = ep - e
        idx3 = jnp.pad(idx3, ((0, 0), (0, pad)))
        h_s = jnp.pad(h_s, ((0, pad), (0, 0)))
        qt = jnp.pad(qt, ((0, 0), (0, pad)))
    nb = ep // E_BLOCK

    body = functools.partial(_hgt_kernel, maxlen=maxlen, nn=nn, ne=ne,
                             h=h, dk=dk)
    col_spec = lambda r: pl.BlockSpec((r, E_BLOCK), lambda i: (0, i))
    full_spec = lambda a: pl.BlockSpec(a.shape, lambda i: (0,) * a.ndim)
    att_t, m_t = pl.pallas_call(
        body,
        grid=(nb,),
        in_specs=[
            col_spec(3),
            pl.BlockSpec((E_BLOCK, d_in), lambda i: (i, 0)),
            col_spec(h * dk),
            full_spec(table16),
            full_spec(kcat16),
            full_spec(vcat16),
            full_spec(kb_t),
            full_spec(vb_t),
            full_spec(wmsg16),
            full_spec(watt16),
            full_spec(mu_t),
            full_spec(seg),
        ],
        out_specs=[
            pl.BlockSpec((h, E_BLOCK), lambda i: (0, i)),
            pl.BlockSpec((d_out, E_BLOCK), lambda i: (0, i)),
        ],
        out_shape=[
            jax.ShapeDtypeStruct((h, ep), jnp.float32),
            jax.ShapeDtypeStruct((d_out, ep), jnp.float32),
        ],
        compiler_params=pltpu.CompilerParams(
            dimension_semantics=("parallel",)),
    )(idx3, h_s, qt, table16, kcat16, vcat16, kb_t, vb_t, wmsg16, watt16,
      mu_t, seg)

    att = att_t[:, :e].T
    m = m_t[:, :e].reshape(h, dk, e).transpose(2, 0, 1)
    return att, m
